# baseline (speedup 1.0000x reference)
import jax
import jax.numpy as jnp
from jax.experimental import pallas as pl

BETA1 = 0.9
BETA2 = 0.999
EPS = 1e-08
LR = 0.001

_R = 1000


def _adam_body(cnt_ref, gsum_ref, step_ref, emb_ref, mem_ref, pow_ref,
               oemb_ref, ostep_ref, omem_ref, opow_ref):
    cnt = cnt_ref[...]
    step0 = step_ref[...]
    gsum = gsum_ref[...]
    emb = emb_ref[...]
    mem = mem_ref[...]
    power = pow_ref[...]

    mask = cnt > 0.0
    gv = gsum / jnp.where(mask, cnt, 1.0)
    step = step0 + 1.0
    upd_mem = BETA1 * mem + (1.0 - BETA1) * gv
    upd_pow = BETA2 * power + (1.0 - BETA2) * (gv * gv)
    c1 = 1.0 - jnp.exp(step * jnp.log(BETA1).astype(jnp.float32))
    c2 = 1.0 - jnp.exp(step * jnp.log(BETA2).astype(jnp.float32))
    mem_corr = upd_mem / c1
    pow_corr = upd_pow / c2
    std = LR * mem_corr / (jnp.sqrt(pow_corr) + EPS)
    upd_emb = emb - std

    oemb_ref[...] = jnp.where(mask, upd_emb, emb)
    ostep_ref[...] = jnp.where(mask, step, step0)
    omem_ref[...] = jnp.where(mask, upd_mem, mem)
    opow_ref[...] = jnp.where(mask, upd_pow, power)


def kernel(idx, grad, emb, state_step, state_mem, state_power):
    M, D = emb.shape
    cnt = jnp.zeros((M,), jnp.float32).at[idx].add(1.0)
    gsum = jnp.zeros((M, D), jnp.float32).at[idx].add(grad)

    cnt2 = cnt.reshape(M, 1)
    step2 = state_step.reshape(M, 1)

    grid = (M // _R,)
    row_spec = pl.BlockSpec((_R, D), lambda i: (i, 0))
    col_spec = pl.BlockSpec((_R, 1), lambda i: (i, 0))

    oemb, ostep, omem, opow = pl.pallas_call(
        _adam_body,
        grid=grid,
        in_specs=[col_spec, row_spec, col_spec, row_spec, row_spec, row_spec],
        out_specs=[row_spec, col_spec, row_spec, row_spec],
        out_shape=[
            jax.ShapeDtypeStruct((M, D), jnp.float32),
            jax.ShapeDtypeStruct((M, 1), jnp.float32),
            jax.ShapeDtypeStruct((M, D), jnp.float32),
            jax.ShapeDtypeStruct((M, D), jnp.float32),
        ],
    )(cnt2, gsum, step2, emb, state_mem, state_power)

    return oemb, ostep.reshape(M), omem, opow
